# baseline (device time: 76384 ns/iter reference)
import jax
import jax.numpy as jnp
from jax import lax
from jax.experimental import pallas as pl
from jax.experimental.pallas import tpu as pltpu

N_DEV = 4
SQ = 1024
SKV = 1024
HQ = 8
DH = 128
D = HQ * DH
BLK = 64
QW = SQ // N_DEV
SCALE = 0.08838834764831843

ROWS = (768, 512, 0, 256)
PREFIX = (1024, 768, 256, 512)


def _body(x_ref, wq_ref, kv_ref, wo_ref, out_ref,
          comm_ref, ctx_ref, mine_ref,
          kv_ssems, kv_rsems, g_ssems, g_rsems):
    my = lax.axis_index("i")

    def mk_kv(h, tgt_idx, tgt):
        p = PREFIX[tgt]
        return pltpu.make_async_remote_copy(
            src_ref=comm_ref.at[h, :, pl.ds(0, p), :],
            dst_ref=comm_ref.at[h, :, pl.ds(0, p), :],
            send_sem=kv_ssems.at[tgt_idx, h],
            recv_sem=kv_rsems.at[h],
            device_id=(tgt,),
            device_id_type=pl.DeviceIdType.MESH,
        )

    to = {t: [mk_kv(h, i, t) for h in range(HQ)]
          for i, t in enumerate((1, 3, 2))}

    @pl.when(my == 0)
    def _():
        comm_ref[...] = kv_ref[...]
        for h in range(HQ):
            to[1][h].start()
        for h in range(HQ):
            to[2][h].start()
        for h in range(HQ):
            to[3][h].start()

    for d in (1, 2, 3):
        @pl.when(my == d)
        def _(d=d):
            comm_ref[:, 1, PREFIX[d]:, :] = jnp.zeros(
                (HQ, SKV - PREFIX[d], DH), jnp.bfloat16)

    start = jnp.where(my == 0, ROWS[0],
                      jnp.where(my == 1, ROWS[1],
                                jnp.where(my == 2, ROWS[2], ROWS[3])))

    xq = x_ref[0, pl.ds(start, QW), :].astype(jnp.bfloat16)
    wqb = wq_ref[...].astype(jnp.bfloat16)
    qq = jnp.dot(xq, wqb,
                 preferred_element_type=jnp.float32).astype(jnp.bfloat16)

    qb = (start + lax.broadcasted_iota(jnp.int32, (QW, SKV), 0)) // BLK
    kb = lax.broadcasted_iota(jnp.int32, (QW, SKV), 1) // BLK
    mask = kb <= qb

    for h in range(HQ):
        for d in (1, 2, 3):
            @pl.when(my == d)
            def _(h=h, d=d):
                to[d][h].wait_recv()

        k = comm_ref[h, 0]
        v = comm_ref[h, 1]
        qh = qq[:, h * DH:(h + 1) * DH]
        s = lax.dot_general(
            qh, k, (((1,), (1,)), ((), ())),
            preferred_element_type=jnp.float32,
        ) * SCALE
        w = jnp.where(mask, jnp.exp(s), 0.0)
        p = (w / jnp.sum(w, axis=1, keepdims=True)).astype(jnp.bfloat16)
        ctx = jnp.dot(p, v, preferred_element_type=jnp.float32)
        ctx_ref[:, h * DH:(h + 1) * DH] = ctx.astype(jnp.bfloat16)

    wob = wo_ref[...].astype(jnp.bfloat16)
    myout = jnp.dot(ctx_ref[...], wob, preferred_element_type=jnp.float32)
    mine_ref[...] = myout
    out_ref[pl.ds(start, QW), :] = myout

    gath = {}
    for o in range(N_DEV):
        others = [t for t in range(N_DEV) if t != o]
        gath[o] = [
            pltpu.make_async_remote_copy(
                src_ref=mine_ref,
                dst_ref=out_ref.at[pl.ds(ROWS[o], QW), :],
                send_sem=g_ssems.at[i],
                recv_sem=g_rsems.at[o],
                device_id=(t,),
                device_id_type=pl.DeviceIdType.MESH,
            )
            for i, t in enumerate(others)
        ]

    for o in range(N_DEV):
        @pl.when(my == o)
        def _(o=o):
            for gd in gath[o]:
                gd.start()

    @pl.when(my == 0)
    def _():
        for t in (1, 2, 3):
            for h in range(HQ):
                to[t][h].wait_send()

    for o in range(N_DEV):
        @pl.when(my == o)
        def _(o=o):
            for gd in gath[o]:
                gd.wait_send()

        @pl.when(my != o)
        def _(o=o):
            gath[o][0].wait_recv()


def kernel(x, Wq, K_ext, V_ext, Wo):
    bf16 = jnp.bfloat16
    kvb = jnp.stack(
        [K_ext[0].astype(bf16).transpose(1, 0, 2),
         V_ext[0].astype(bf16).transpose(1, 0, 2)],
        axis=1,
    )

    out = pl.pallas_call(
        _body,
        out_shape=jax.ShapeDtypeStruct((SQ, D), jnp.float32),
        in_specs=[pl.BlockSpec(memory_space=pltpu.VMEM)] * 4,
        out_specs=pl.BlockSpec(memory_space=pltpu.VMEM),
        scratch_shapes=[
            pltpu.VMEM((HQ, 2, SKV, DH), bf16),
            pltpu.VMEM((QW, D), bf16),
            pltpu.VMEM((QW, D), jnp.float32),
            pltpu.SemaphoreType.DMA((3, HQ)),
            pltpu.SemaphoreType.DMA((HQ,)),
            pltpu.SemaphoreType.DMA((3,)),
            pltpu.SemaphoreType.DMA((N_DEV,)),
        ],
    )(x, Wq, kvb, Wo)

    return out.reshape(1, SQ, D)
